# baseline (speedup 1.0000x reference)
import functools

import jax
import jax.numpy as jnp
from jax import lax
from jax.experimental import pallas as pl
from jax.experimental.pallas import tpu as pltpu, tpu_sc as plsc

EPS = 1e-05


def _emb_body(usage_ref, esum_ref, emb_ref, emb2_ref, e2_ref):
    u = jnp.clip(usage_ref[...], EPS, None)
    emb = esum_ref[...] / u
    emb_ref[...] = emb
    emb2_ref[...] = emb + emb
    e2_ref[...] = jnp.sum(emb * emb, axis=1, keepdims=True)


def _compute_emb(cluster_usage, embedding_sum, tk=2048):
    K, D = embedding_sum.shape
    usage2d = cluster_usage.reshape(K, 1)
    emb, emb2, e2 = pl.pallas_call(
        _emb_body,
        grid=(K // tk,),
        in_specs=[
            pl.BlockSpec((tk, 1), lambda i: (i, 0)),
            pl.BlockSpec((tk, D), lambda i: (i, 0)),
        ],
        out_specs=[
            pl.BlockSpec((tk, D), lambda i: (i, 0)),
            pl.BlockSpec((tk, D), lambda i: (i, 0)),
            pl.BlockSpec((tk, 1), lambda i: (i, 0)),
        ],
        out_shape=[
            jax.ShapeDtypeStruct((K, D), jnp.float32),
            jax.ShapeDtypeStruct((K, D), jnp.float32),
            jax.ShapeDtypeStruct((K, 1), jnp.float32),
        ],
    )(usage2d, embedding_sum)
    return emb, emb2, e2


def _argmin_body(x_ref, emb2_ref, e2_ref, iota_ref, codes_ref):
    xb = x_ref[...]
    x2 = jnp.sum(xb * xb, axis=1, keepdims=True)
    dot2 = lax.dot_general(xb, emb2_ref[...], (((1,), (1,)), ((), ())),
                           preferred_element_type=jnp.float32)
    d2 = (x2 + e2_ref[...]) - dot2
    m = jnp.min(d2, axis=1, keepdims=True)
    s = jnp.sqrt(m)
    nxt = lax.bitcast_convert_type(
        lax.bitcast_convert_type(s, jnp.int32) + 1, jnp.float32)
    ulp = nxt - s
    hu = 0.5 * ulp
    c = s * 4097.0
    hi = c - (c - s)
    lo = s - hi
    p1 = s * s
    e1 = ((hi * hi - p1) + 2.0 * (hi * lo)) + lo * lo
    r = (e1 + s * ulp) + hu * hu
    q = p1 + r
    dq = (p1 - q) + r
    qprev = lax.bitcast_convert_type(
        lax.bitcast_convert_type(q, jnp.int32) - 1, jnp.float32)
    u = jnp.maximum(jnp.where(dq >= 0.0, q, qprev), m)

    cand = jnp.where(d2 <= u, iota_ref[...], jnp.float32(3e38))
    rowarg = jnp.min(cand, axis=1, keepdims=True)
    codes_ref[...] = rowarg.astype(jnp.int32)


def _compute_codes(x_part, emb2, e2_row, iota_row, tn=256):
    N, D = x_part.shape
    K = emb2.shape[0]
    codes = pl.pallas_call(
        _argmin_body,
        grid=(N // tn,),
        in_specs=[
            pl.BlockSpec((tn, D), lambda n: (n, 0)),
            pl.BlockSpec((K, D), lambda n: (0, 0)),
            pl.BlockSpec((1, K), lambda n: (0, 0)),
            pl.BlockSpec((1, K), lambda n: (0, 0)),
        ],
        out_specs=pl.BlockSpec((tn, 1), lambda n: (n, 0)),
        out_shape=jax.ShapeDtypeStruct((N, 1), jnp.int32),
        compiler_params=pltpu.CompilerParams(
            dimension_semantics=("parallel",),
        ),
    )(x_part, emb2, e2_row, iota_row)
    return codes.reshape(N)


_SC_CHUNK = 128
_SC_NBUF = 3


def _sc_gather(emb, codes):
    N, = codes.shape
    K, D = emb.shape
    info = plsc.get_sparse_core_info()
    nc, ns = info.num_cores, info.num_subcores
    nw = nc * ns
    per_w = N // nw
    n_chunks = per_w // _SC_CHUNK
    nbuf = min(_SC_NBUF, n_chunks)
    mesh = plsc.VectorSubcoreMesh(core_axis_name="c", subcore_axis_name="s")

    @functools.partial(
        pl.kernel,
        mesh=mesh,
        out_type=jax.ShapeDtypeStruct((N, D), jnp.float32),
        scratch_types=(
            [pltpu.VMEM((per_w,), jnp.int32)]
            + [pltpu.VMEM((_SC_CHUNK, D), jnp.float32)] * nbuf
            + [pltpu.SemaphoreType.DMA] * (2 * nbuf)
        ),
    )
    def gather_k(emb_hbm, codes_hbm, out_hbm, idx_v, *bufsem):
        bufs = bufsem[:nbuf]
        gsems = bufsem[nbuf:2 * nbuf]
        wsems = bufsem[2 * nbuf:]
        wid = lax.axis_index("s") * nc + lax.axis_index("c")
        base = wid * per_w
        pltpu.sync_copy(codes_hbm.at[pl.ds(base, per_w)], idx_v)

        def start_gather(c):
            pltpu.async_copy(
                emb_hbm.at[idx_v.at[pl.ds(c * _SC_CHUNK, _SC_CHUNK)]],
                bufs[c % nbuf], gsems[c % nbuf])

        for c in range(min(nbuf, n_chunks)):
            start_gather(c)
        for c in range(n_chunks):
            b = c % nbuf
            pltpu.make_async_copy(
                emb_hbm.at[idx_v.at[pl.ds(c * _SC_CHUNK, _SC_CHUNK)]],
                bufs[b], gsems[b]).wait()
            wcopy = pltpu.async_copy(
                bufs[b], out_hbm.at[pl.ds(base + c * _SC_CHUNK, _SC_CHUNK)],
                wsems[b])
            if c + nbuf < n_chunks:
                wcopy.wait()
                start_gather(c + nbuf)
        for c in range(max(0, n_chunks - nbuf), n_chunks):
            b = c % nbuf
            pltpu.make_async_copy(
                bufs[b],
                out_hbm.at[pl.ds(base + c * _SC_CHUNK, _SC_CHUNK)],
                wsems[b]).wait()

    return gather_k(emb, codes)


def kernel(x, cluster_usage, embedding_sum):
    B, D, T = x.shape
    K = embedding_sum.shape[0]
    N = B * T
    emb, emb2, e2 = _compute_emb(cluster_usage, embedding_sum)
    e2_row = e2.reshape(1, K)
    iota_row = lax.broadcasted_iota(jnp.float32, (1, K), 1)
    x_flat = jnp.transpose(x, (0, 2, 1)).reshape(N, D)

    halves = []
    bh = B // 2
    for h in range(2):
        x_part = lax.slice(x_flat, (h * (N // 2), 0), ((h + 1) * (N // 2), D))
        codes = _compute_codes(x_part, emb2, e2_row, iota_row)
        dec = _sc_gather(emb, codes)
        halves.append(jnp.transpose(dec.reshape(bh, T, D), (0, 2, 1)))
    return jnp.concatenate(halves, axis=0)


# speedup vs baseline: 1.0776x; 1.0776x over previous
import functools

import jax
import jax.numpy as jnp
from jax import lax
from jax.experimental import pallas as pl
from jax.experimental.pallas import tpu as pltpu, tpu_sc as plsc

EPS = 1e-05


def _emb_body(usage_ref, esum_ref, emb_ref, emb2_ref, e2_ref):
    u = jnp.clip(usage_ref[...], EPS, None)
    emb = esum_ref[...] / u
    emb_ref[...] = emb
    emb2_ref[...] = emb + emb
    e2_ref[...] = jnp.sum(emb * emb, axis=1, keepdims=True)


def _compute_emb(cluster_usage, embedding_sum, tk=2048):
    K, D = embedding_sum.shape
    usage2d = cluster_usage.reshape(K, 1)
    emb, emb2, e2 = pl.pallas_call(
        _emb_body,
        grid=(K // tk,),
        in_specs=[
            pl.BlockSpec((tk, 1), lambda i: (i, 0)),
            pl.BlockSpec((tk, D), lambda i: (i, 0)),
        ],
        out_specs=[
            pl.BlockSpec((tk, D), lambda i: (i, 0)),
            pl.BlockSpec((tk, D), lambda i: (i, 0)),
            pl.BlockSpec((tk, 1), lambda i: (i, 0)),
        ],
        out_shape=[
            jax.ShapeDtypeStruct((K, D), jnp.float32),
            jax.ShapeDtypeStruct((K, D), jnp.float32),
            jax.ShapeDtypeStruct((K, 1), jnp.float32),
        ],
    )(usage2d, embedding_sum)
    return emb, emb2, e2


def _argmin_body(x_ref, emb2_ref, e2_ref, iota_ref, codes_ref):
    xb = x_ref[...]
    x2 = jnp.sum(xb * xb, axis=1, keepdims=True)
    dot2 = lax.dot_general(xb, emb2_ref[...], (((1,), (1,)), ((), ())),
                           preferred_element_type=jnp.float32)
    d2 = (x2 + e2_ref[...]) - dot2
    m = jnp.min(d2, axis=1, keepdims=True)
    s = jnp.sqrt(m)
    nxt = lax.bitcast_convert_type(
        lax.bitcast_convert_type(s, jnp.int32) + 1, jnp.float32)
    ulp = nxt - s
    hu = 0.5 * ulp
    c = s * 4097.0
    hi = c - (c - s)
    lo = s - hi
    p1 = s * s
    e1 = ((hi * hi - p1) + 2.0 * (hi * lo)) + lo * lo
    r = (e1 + s * ulp) + hu * hu
    q = p1 + r
    dq = (p1 - q) + r
    qprev = lax.bitcast_convert_type(
        lax.bitcast_convert_type(q, jnp.int32) - 1, jnp.float32)
    u = jnp.maximum(jnp.where(dq >= 0.0, q, qprev), m)

    cand = jnp.where(d2 <= u, iota_ref[...], jnp.float32(3e38))
    rowarg = jnp.min(cand, axis=1, keepdims=True)
    codes_ref[...] = rowarg.astype(jnp.int32)


def _compute_codes(x_part, emb2, e2_row, iota_row, tn=256):
    N, D = x_part.shape
    K = emb2.shape[0]
    codes = pl.pallas_call(
        _argmin_body,
        grid=(N // tn,),
        in_specs=[
            pl.BlockSpec((tn, D), lambda n: (n, 0)),
            pl.BlockSpec((K, D), lambda n: (0, 0)),
            pl.BlockSpec((1, K), lambda n: (0, 0)),
            pl.BlockSpec((1, K), lambda n: (0, 0)),
        ],
        out_specs=pl.BlockSpec((tn, 1), lambda n: (n, 0)),
        out_shape=jax.ShapeDtypeStruct((N, 1), jnp.int32),
        compiler_params=pltpu.CompilerParams(
            dimension_semantics=("parallel",),
        ),
    )(x_part, emb2, e2_row, iota_row)
    return codes.reshape(N)


_SC_CHUNK = 128
_SC_NBUF = 3


def _sc_gather(emb, codes):
    N, = codes.shape
    K, D = emb.shape
    info = plsc.get_sparse_core_info()
    nc, ns = info.num_cores, info.num_subcores
    nw = nc * ns
    per_w = N // nw
    n_chunks = per_w // _SC_CHUNK
    nbuf = min(_SC_NBUF, n_chunks)
    mesh = plsc.VectorSubcoreMesh(core_axis_name="c", subcore_axis_name="s")

    @functools.partial(
        pl.kernel,
        mesh=mesh,
        out_type=jax.ShapeDtypeStruct((N, D), jnp.float32),
        scratch_types=(
            [pltpu.VMEM((per_w,), jnp.int32)]
            + [pltpu.VMEM((_SC_CHUNK, D), jnp.float32)] * nbuf
            + [pltpu.SemaphoreType.DMA] * (2 * nbuf)
        ),
    )
    def gather_k(emb_hbm, codes_hbm, out_hbm, idx_v, *bufsem):
        bufs = bufsem[:nbuf]
        gsems = bufsem[nbuf:2 * nbuf]
        wsems = bufsem[2 * nbuf:]
        wid = lax.axis_index("s") * nc + lax.axis_index("c")
        base = wid * per_w
        pltpu.sync_copy(codes_hbm.at[pl.ds(base, per_w)], idx_v)

        def start_gather(c):
            pltpu.async_copy(
                emb_hbm.at[idx_v.at[pl.ds(c * _SC_CHUNK, _SC_CHUNK)]],
                bufs[c % nbuf], gsems[c % nbuf])

        for c in range(min(nbuf, n_chunks)):
            start_gather(c)
        for c in range(n_chunks):
            b = c % nbuf
            pltpu.make_async_copy(
                emb_hbm.at[idx_v.at[pl.ds(c * _SC_CHUNK, _SC_CHUNK)]],
                bufs[b], gsems[b]).wait()
            wcopy = pltpu.async_copy(
                bufs[b], out_hbm.at[pl.ds(base + c * _SC_CHUNK, _SC_CHUNK)],
                wsems[b])
            if c + nbuf < n_chunks:
                wcopy.wait()
                start_gather(c + nbuf)
        for c in range(max(0, n_chunks - nbuf), n_chunks):
            b = c % nbuf
            pltpu.make_async_copy(
                bufs[b],
                out_hbm.at[pl.ds(base + c * _SC_CHUNK, _SC_CHUNK)],
                wsems[b]).wait()

    return gather_k(emb, codes)


def kernel(x, cluster_usage, embedding_sum):
    B, D, T = x.shape
    K = embedding_sum.shape[0]
    N = B * T
    emb, emb2, e2 = _compute_emb(cluster_usage, embedding_sum)
    e2_row = e2.reshape(1, K)
    iota_row = lax.broadcasted_iota(jnp.float32, (1, K), 1)
    x_flat = jnp.transpose(x, (0, 2, 1)).reshape(N, D)
    codes = _compute_codes(x_flat, emb2, e2_row, iota_row)
    dec = _sc_gather(emb, codes)
    return jnp.transpose(dec.reshape(B, T, D), (0, 2, 1))


# speedup vs baseline: 1.0805x; 1.0027x over previous
import functools

import jax
import jax.numpy as jnp
from jax import lax
from jax.experimental import pallas as pl
from jax.experimental.pallas import tpu as pltpu, tpu_sc as plsc

EPS = 1e-05


def _emb_body(usage_ref, esum_ref, emb_ref, emb2_ref, e2_ref):
    u = jnp.clip(usage_ref[...], EPS, None)
    emb = esum_ref[...] / u
    emb_ref[...] = emb
    emb2_ref[...] = emb + emb
    e2_ref[...] = jnp.sum(emb * emb, axis=1, keepdims=True)


def _compute_emb(cluster_usage, embedding_sum, tk=2048):
    K, D = embedding_sum.shape
    usage2d = cluster_usage.reshape(K, 1)
    emb, emb2, e2 = pl.pallas_call(
        _emb_body,
        grid=(K // tk,),
        in_specs=[
            pl.BlockSpec((tk, 1), lambda i: (i, 0)),
            pl.BlockSpec((tk, D), lambda i: (i, 0)),
        ],
        out_specs=[
            pl.BlockSpec((tk, D), lambda i: (i, 0)),
            pl.BlockSpec((tk, D), lambda i: (i, 0)),
            pl.BlockSpec((tk, 1), lambda i: (i, 0)),
        ],
        out_shape=[
            jax.ShapeDtypeStruct((K, D), jnp.float32),
            jax.ShapeDtypeStruct((K, D), jnp.float32),
            jax.ShapeDtypeStruct((K, 1), jnp.float32),
        ],
    )(usage2d, embedding_sum)
    return emb, emb2, e2


def _argmin_body(x_ref, emb2_ref, e2_ref, iota_ref, codes_ref):
    xb = x_ref[...]
    x2 = jnp.sum(xb * xb, axis=1, keepdims=True)
    dot2 = lax.dot_general(xb, emb2_ref[...], (((1,), (1,)), ((), ())),
                           preferred_element_type=jnp.float32)
    d2 = (x2 + e2_ref[...]) - dot2
    m = jnp.min(d2, axis=1, keepdims=True)
    s = jnp.sqrt(m)
    nxt = lax.bitcast_convert_type(
        lax.bitcast_convert_type(s, jnp.int32) + 1, jnp.float32)
    ulp = nxt - s
    hu = 0.5 * ulp
    c = s * 4097.0
    hi = c - (c - s)
    lo = s - hi
    p1 = s * s
    e1 = ((hi * hi - p1) + 2.0 * (hi * lo)) + lo * lo
    r = (e1 + s * ulp) + hu * hu
    q = p1 + r
    dq = (p1 - q) + r
    qprev = lax.bitcast_convert_type(
        lax.bitcast_convert_type(q, jnp.int32) - 1, jnp.float32)
    u = jnp.maximum(jnp.where(dq >= 0.0, q, qprev), m)

    cand = jnp.where(d2 <= u, iota_ref[...], jnp.float32(3e38))
    rowarg = jnp.min(cand, axis=1, keepdims=True)
    codes_ref[...] = rowarg.astype(jnp.int32)


def _compute_codes(x_part, emb2, e2_row, iota_row, tn=256):
    N, D = x_part.shape
    K = emb2.shape[0]
    codes = pl.pallas_call(
        _argmin_body,
        grid=(N // tn,),
        in_specs=[
            pl.BlockSpec((tn, D), lambda n: (n, 0)),
            pl.BlockSpec((K, D), lambda n: (0, 0)),
            pl.BlockSpec((1, K), lambda n: (0, 0)),
            pl.BlockSpec((1, K), lambda n: (0, 0)),
        ],
        out_specs=pl.BlockSpec((tn, 1), lambda n: (n, 0)),
        out_shape=jax.ShapeDtypeStruct((N, 1), jnp.int32),
        compiler_params=pltpu.CompilerParams(
            dimension_semantics=("arbitrary",),
        ),
    )(x_part, emb2, e2_row, iota_row)
    return codes.reshape(N)


_SC_CHUNK = 128
_SC_NBUF = 3


def _sc_gather(emb, codes):
    N, = codes.shape
    K, D = emb.shape
    info = plsc.get_sparse_core_info()
    nc, ns = info.num_cores, info.num_subcores
    nw = nc * ns
    per_w = N // nw
    n_chunks = per_w // _SC_CHUNK
    nbuf = min(_SC_NBUF, n_chunks)
    mesh = plsc.VectorSubcoreMesh(core_axis_name="c", subcore_axis_name="s")

    @functools.partial(
        pl.kernel,
        mesh=mesh,
        out_type=jax.ShapeDtypeStruct((N, D), jnp.float32),
        scratch_types=(
            [pltpu.VMEM((per_w,), jnp.int32)]
            + [pltpu.VMEM((_SC_CHUNK, D), jnp.float32)] * nbuf
            + [pltpu.SemaphoreType.DMA] * (2 * nbuf)
        ),
    )
    def gather_k(emb_hbm, codes_hbm, out_hbm, idx_v, *bufsem):
        bufs = bufsem[:nbuf]
        gsems = bufsem[nbuf:2 * nbuf]
        wsems = bufsem[2 * nbuf:]
        wid = lax.axis_index("s") * nc + lax.axis_index("c")
        base = wid * per_w
        pltpu.sync_copy(codes_hbm.at[pl.ds(base, per_w)], idx_v)

        def start_gather(c):
            pltpu.async_copy(
                emb_hbm.at[idx_v.at[pl.ds(c * _SC_CHUNK, _SC_CHUNK)]],
                bufs[c % nbuf], gsems[c % nbuf])

        for c in range(min(nbuf, n_chunks)):
            start_gather(c)
        for c in range(n_chunks):
            b = c % nbuf
            pltpu.make_async_copy(
                emb_hbm.at[idx_v.at[pl.ds(c * _SC_CHUNK, _SC_CHUNK)]],
                bufs[b], gsems[b]).wait()
            wcopy = pltpu.async_copy(
                bufs[b], out_hbm.at[pl.ds(base + c * _SC_CHUNK, _SC_CHUNK)],
                wsems[b])
            if c + nbuf < n_chunks:
                wcopy.wait()
                start_gather(c + nbuf)
        for c in range(max(0, n_chunks - nbuf), n_chunks):
            b = c % nbuf
            pltpu.make_async_copy(
                bufs[b],
                out_hbm.at[pl.ds(base + c * _SC_CHUNK, _SC_CHUNK)],
                wsems[b]).wait()

    return gather_k(emb, codes)


def kernel(x, cluster_usage, embedding_sum):
    B, D, T = x.shape
    K = embedding_sum.shape[0]
    N = B * T
    emb, emb2, e2 = _compute_emb(cluster_usage, embedding_sum)
    e2_row = e2.reshape(1, K)
    iota_row = lax.broadcasted_iota(jnp.float32, (1, K), 1)
    x_flat = jnp.transpose(x, (0, 2, 1)).reshape(N, D)
    codes = _compute_codes(x_flat, emb2, e2_row, iota_row)
    dec = _sc_gather(emb, codes)
    return jnp.transpose(dec.reshape(B, T, D), (0, 2, 1))
